# Initial kernel scaffold; baseline (speedup 1.0000x reference)
import jax
import jax.numpy as jnp
from jax.experimental import pallas as pl


def kernel(x, edge_index, W, b):
    raise NotImplementedError("write your pallas kernel here")


# speedup vs baseline: 28.5224x; 28.5224x over previous
import functools

import jax
import jax.numpy as jnp
from jax import lax
from jax.experimental import pallas as pl
from jax.experimental.pallas import tpu as pltpu
from jax.experimental.pallas import tpu_sc as plsc

N_NODES = 10000
N_EDGES = 320000
D = 128
NC = 2
NS = 16
NW = NC * NS
N_PAD = 10240
EPT = N_EDGES // NW
CH = 128
NCHUNK = 80
EPT_PAD = NCHUNK * CH
ROWS_PT = N_PAD // NS

_f32 = jnp.float32


def _deg_body(dst_hbm, deg_hbm, deg_sh, idx_v, ones_v, zbuf):
    c = lax.axis_index("c")
    s = lax.axis_index("s")
    wid = c * NS + s

    def z16(i, _):
        zbuf[pl.ds(i * 16, 16)] = jnp.zeros((16,), _f32)
        return 0

    lax.fori_loop(0, ROWS_PT // 16, z16, 0)
    pltpu.sync_copy(zbuf, deg_sh.at[pl.ds(s * ROWS_PT, ROWS_PT)])

    def o16(i, _):
        ones_v[pl.ds(i * 16, 16)] = jnp.ones((16,), _f32)
        return 0

    lax.fori_loop(0, CH // 16, o16, 0)
    pltpu.sync_copy(dst_hbm.at[wid], idx_v)
    plsc.subcore_barrier()

    def chunk(j, _):
        pltpu.sync_copy(ones_v, deg_sh.at[idx_v.at[j]], add=True)
        return 0

    lax.fori_loop(0, NCHUNK, chunk, 0)
    plsc.subcore_barrier()
    pltpu.sync_copy(deg_sh.at[pl.ds(s * ROWS_PT, ROWS_PT)],
                    deg_hbm.at[c, pl.ds(s * ROWS_PT, ROWS_PT)])


def _mm_body(deg_ref, x_ref, w_ref, y_ref, dis_ref):
    deg = deg_ref[0, :] + deg_ref[1, :]
    dis = jnp.where(deg > 0, lax.rsqrt(jnp.maximum(deg, 1e-12)), 0.0)
    xw = jnp.dot(x_ref[...], w_ref[...], preferred_element_type=_f32)
    y = xw * dis[:N_NODES, None]
    y_ref[...] = jnp.concatenate(
        [y, jnp.zeros((N_PAD - N_NODES, D), _f32)], axis=0)
    dis_ref[...] = dis


def _edge_body(y_hbm, src_hbm, dst_hbm, part_hbm, acc_sh, src_v, dst_v,
               rows_v, sem_g):
    c = lax.axis_index("c")
    s = lax.axis_index("s")
    wid = c * NS + s

    def z(i, _):
        rows_v[i // 8, pl.ds((i % 8) * 16, 16)] = jnp.zeros((16,), _f32)
        return 0

    lax.fori_loop(0, CH * 8, z, 0)

    def zacc(k, _):
        pltpu.sync_copy(rows_v, acc_sh.at[pl.ds(s * ROWS_PT + k * CH, CH)])
        return 0

    lax.fori_loop(0, ROWS_PT // CH, zacc, 0)
    pltpu.sync_copy(src_hbm.at[wid], src_v)
    pltpu.sync_copy(dst_hbm.at[wid], dst_v)
    plsc.subcore_barrier()

    def chunk(j, _):
        pltpu.async_copy(y_hbm.at[src_v.at[j]], rows_v, sem_g).wait()
        pltpu.sync_copy(rows_v, acc_sh.at[dst_v.at[j]], add=True)
        return 0

    lax.fori_loop(0, NCHUNK, chunk, 0)
    plsc.subcore_barrier()

    def out_cp(k, _):
        pltpu.sync_copy(acc_sh.at[pl.ds(s * ROWS_PT + k * CH, CH)],
                        part_hbm.at[c, pl.ds(s * ROWS_PT + k * CH, CH)])
        return 0

    lax.fori_loop(0, ROWS_PT // CH, out_cp, 0)


def _fin_body(part_ref, dis_ref, b_ref, out_ref):
    p = part_ref[0, :N_NODES, :] + part_ref[1, :N_NODES, :]
    dis = dis_ref[...]
    out_ref[...] = p * dis[:N_NODES, None] + b_ref[...][None, :]


@functools.lru_cache(maxsize=1)
def _build():
    mesh = plsc.VectorSubcoreMesh(core_axis_name="c", subcore_axis_name="s",
                                  num_cores=NC, num_subcores=NS)
    deg_call = pl.kernel(
        _deg_body,
        out_type=jax.ShapeDtypeStruct((NC, N_PAD), _f32),
        mesh=mesh,
        scratch_types=[
            pltpu.VMEM_SHARED((N_PAD,), _f32),
            pltpu.VMEM((NCHUNK, CH), jnp.int32),
            pltpu.VMEM((CH,), _f32),
            pltpu.VMEM((ROWS_PT,), _f32),
        ],
    )
    mm_call = pl.pallas_call(
        _mm_body,
        out_shape=[jax.ShapeDtypeStruct((N_PAD, D), _f32),
                   jax.ShapeDtypeStruct((N_PAD,), _f32)],
    )
    edge_call = pl.kernel(
        _edge_body,
        out_type=jax.ShapeDtypeStruct((NC, N_PAD, D), _f32),
        mesh=mesh,
        scratch_types=[
            pltpu.VMEM_SHARED((N_PAD, D), _f32),
            pltpu.VMEM((NCHUNK, CH), jnp.int32),
            pltpu.VMEM((NCHUNK, CH), jnp.int32),
            pltpu.VMEM((CH, D), _f32),
            pltpu.SemaphoreType.DMA,
        ],
    )
    fin_call = pl.pallas_call(
        _fin_body,
        out_shape=jax.ShapeDtypeStruct((N_NODES, D), _f32),
    )
    return deg_call, mm_call, edge_call, fin_call


def kernel(x, edge_index, W, b):
    deg_call, mm_call, edge_call, fin_call = _build()
    src = edge_index[0].astype(jnp.int32)
    dst = edge_index[1].astype(jnp.int32)
    padi = N_NODES + (jnp.arange(EPT_PAD - EPT, dtype=jnp.int32)
                      % (N_PAD - N_NODES))
    pad_t = jnp.broadcast_to(padi, (NW, EPT_PAD - EPT))
    src_r = jnp.concatenate([src.reshape(NW, EPT), pad_t],
                            axis=1).reshape(NW, NCHUNK, CH)
    dst_r = jnp.concatenate([dst.reshape(NW, EPT), pad_t],
                            axis=1).reshape(NW, NCHUNK, CH)
    deg_p = deg_call(dst_r)
    y, dis = mm_call(deg_p, x, W)
    part = edge_call(y, src_r, dst_r)
    out = fin_call(part, dis, b)
    return (out, edge_index)


# speedup vs baseline: 36.9077x; 1.2940x over previous
import functools

import jax
import jax.numpy as jnp
from jax import lax
from jax.experimental import pallas as pl
from jax.experimental.pallas import tpu as pltpu
from jax.experimental.pallas import tpu_sc as plsc

N_NODES = 10000
N_EDGES = 320000
D = 128
NC = 2
NS = 16
NW = NC * NS
N_PAD = 10240
EPT = N_EDGES // NW
CH = 64
SB = 16
NBLK = 10
NCHUNK = SB * NBLK
EPT_PAD = NCHUNK * CH
CH_D = 128
NCHUNK_D = 80
EPT_PAD_D = NCHUNK_D * CH_D
ROWS_PT = N_PAD // NS
ZC = 32

_f32 = jnp.float32


def _deg_body(dst_hbm, deg_hbm, deg_sh, idx_v, ones_v, zbuf):
    c = lax.axis_index("c")
    s = lax.axis_index("s")
    wid = c * NS + s

    def z16(i, _):
        zbuf[pl.ds(i * 16, 16)] = jnp.zeros((16,), _f32)
        return 0

    lax.fori_loop(0, ROWS_PT // 16, z16, 0)
    pltpu.sync_copy(zbuf, deg_sh.at[pl.ds(s * ROWS_PT, ROWS_PT)])

    def o16(i, _):
        ones_v[pl.ds(i * 16, 16)] = jnp.ones((16,), _f32)
        return 0

    lax.fori_loop(0, CH_D // 16, o16, 0)
    pltpu.sync_copy(dst_hbm.at[wid], idx_v)
    plsc.subcore_barrier()

    def chunk(j, _):
        pltpu.sync_copy(ones_v, deg_sh.at[idx_v.at[j]], add=True)
        return 0

    lax.fori_loop(0, NCHUNK_D, chunk, 0)
    plsc.subcore_barrier()
    pltpu.sync_copy(deg_sh.at[pl.ds(s * ROWS_PT, ROWS_PT)],
                    deg_hbm.at[c, pl.ds(s * ROWS_PT, ROWS_PT)])


def _mm_body(deg_ref, x_ref, w_ref, y_ref, dis_ref):
    deg = deg_ref[0, :] + deg_ref[1, :]
    dis = jnp.where(deg > 0, lax.rsqrt(jnp.maximum(deg, 1e-12)), 0.0)
    xw = jnp.dot(x_ref[...], w_ref[...], preferred_element_type=_f32)
    y = xw * dis[:N_NODES, None]
    y_ref[...] = jnp.concatenate(
        [y, jnp.zeros((N_PAD - N_NODES, D), _f32)], axis=0)
    dis_ref[...] = dis


NBUF = 4


def _edge_body(y_hbm, src_hbm, dst_hbm, part_hbm, acc_sh, srcb, dstb,
               r0, r1, r2, r3, si0, si1, s0, s1, s2, s3):
    rows = (r0, r1, r2, r3)
    sems = (s0, s1, s2, s3)
    si = (si0, si1)
    c = lax.axis_index("c")
    s = lax.axis_index("s")
    wid = c * NS + s

    def z(i, _):
        r0[i // 8, pl.ds((i % 8) * 16, 16)] = jnp.zeros((16,), _f32)
        return 0

    lax.fori_loop(0, ZC * 8, z, 0)

    def zacc(k, _):
        pltpu.sync_copy(r0.at[pl.ds(0, ZC)],
                        acc_sh.at[pl.ds(s * ROWS_PT + k * ZC, ZC)])
        return 0

    lax.fori_loop(0, ROWS_PT // ZC, zacc, 0)
    pltpu.sync_copy(src_hbm.at[wid, 0], srcb.at[0])
    pltpu.sync_copy(dst_hbm.at[wid, 0], dstb.at[0])
    plsc.subcore_barrier()

    def do_block(blk, slot):
        for b in range(NBUF - 1):
            pltpu.async_copy(y_hbm.at[srcb.at[slot, b]], rows[b], sems[b])

        def step(ii, _):
            for b in range(NBUF):
                jl = NBUF * ii + b
                cb = (b + NBUF - 1) % NBUF
                pltpu.make_async_copy(y_hbm.at[srcb.at[slot, jl]], rows[b],
                                      sems[b]).wait()
                pltpu.async_copy(rows[b], acc_sh.at[dstb.at[slot, jl]],
                                 sems[b], add=True)
                @pl.when(jl >= 1)
                def _():
                    pltpu.make_async_copy(rows[cb],
                                          acc_sh.at[dstb.at[slot, jl - 1]],
                                          sems[cb]).wait()

                @pl.when(jl + NBUF - 1 < SB)
                def _():
                    pltpu.async_copy(y_hbm.at[srcb.at[slot, jl + NBUF - 1]],
                                     rows[cb], sems[cb])
            return 0

        lax.fori_loop(0, SB // NBUF, step, 0)
        fb = (SB - 1) % NBUF
        pltpu.make_async_copy(rows[fb], acc_sh.at[dstb.at[slot, SB - 1]],
                              sems[fb]).wait()

    def pair(p, _):
        for slot in range(2):
            blk = 2 * p + slot
            nslot = 1 - slot
            @pl.when(blk >= 1)
            def _():
                pltpu.make_async_copy(src_hbm.at[wid, blk], srcb.at[slot],
                                      si[slot]).wait()
                pltpu.make_async_copy(dst_hbm.at[wid, blk], dstb.at[slot],
                                      si[slot]).wait()

            @pl.when(blk + 1 < NBLK)
            def _():
                pltpu.async_copy(src_hbm.at[wid, blk + 1], srcb.at[nslot],
                                 si[nslot])
                pltpu.async_copy(dst_hbm.at[wid, blk + 1], dstb.at[nslot],
                                 si[nslot])

            do_block(blk, slot)
        return 0

    lax.fori_loop(0, NBLK // 2, pair, 0)
    plsc.subcore_barrier()

    def out_cp(k, _):
        pltpu.sync_copy(acc_sh.at[pl.ds(s * ROWS_PT + k * ZC, ZC)],
                        part_hbm.at[c, pl.ds(s * ROWS_PT + k * ZC, ZC)])
        return 0

    lax.fori_loop(0, ROWS_PT // ZC, out_cp, 0)


def _fin_body(part_ref, dis_ref, b_ref, out_ref):
    p = part_ref[0, :N_NODES, :] + part_ref[1, :N_NODES, :]
    dis = dis_ref[...]
    out_ref[...] = p * dis[:N_NODES, None] + b_ref[...][None, :]


@functools.lru_cache(maxsize=1)
def _build():
    mesh = plsc.VectorSubcoreMesh(core_axis_name="c", subcore_axis_name="s",
                                  num_cores=NC, num_subcores=NS)
    deg_call = pl.kernel(
        _deg_body,
        out_type=jax.ShapeDtypeStruct((NC, N_PAD), _f32),
        mesh=mesh,
        scratch_types=[
            pltpu.VMEM_SHARED((N_PAD,), _f32),
            pltpu.VMEM((NCHUNK_D, CH_D), jnp.int32),
            pltpu.VMEM((CH_D,), _f32),
            pltpu.VMEM((ROWS_PT,), _f32),
        ],
    )
    mm_call = pl.pallas_call(
        _mm_body,
        out_shape=[jax.ShapeDtypeStruct((N_PAD, D), _f32),
                   jax.ShapeDtypeStruct((N_PAD,), _f32)],
    )
    edge_call = pl.kernel(
        _edge_body,
        out_type=jax.ShapeDtypeStruct((NC, N_PAD, D), _f32),
        mesh=mesh,
        scratch_types=[
            pltpu.VMEM_SHARED((N_PAD, D), _f32),
            pltpu.VMEM((2, SB, CH), jnp.int32),
            pltpu.VMEM((2, SB, CH), jnp.int32),
            pltpu.VMEM((CH, D), _f32),
            pltpu.VMEM((CH, D), _f32),
            pltpu.VMEM((CH, D), _f32),
            pltpu.VMEM((CH, D), _f32),
            pltpu.SemaphoreType.DMA,
            pltpu.SemaphoreType.DMA,
            pltpu.SemaphoreType.DMA,
            pltpu.SemaphoreType.DMA,
            pltpu.SemaphoreType.DMA,
            pltpu.SemaphoreType.DMA,
        ],
    )
    fin_call = pl.pallas_call(
        _fin_body,
        out_shape=jax.ShapeDtypeStruct((N_NODES, D), _f32),
    )
    return deg_call, mm_call, edge_call, fin_call


def kernel(x, edge_index, W, b):
    deg_call, mm_call, edge_call, fin_call = _build()
    src = edge_index[0].astype(jnp.int32)
    dst = edge_index[1].astype(jnp.int32)
    def pad_reshape(a, ept_pad, nchunk, ch):
        padi = N_NODES + (jnp.arange(ept_pad - EPT, dtype=jnp.int32)
                          % (N_PAD - N_NODES))
        pad_t = jnp.broadcast_to(padi, (NW, ept_pad - EPT))
        return jnp.concatenate([a.reshape(NW, EPT), pad_t],
                               axis=1).reshape(NW, nchunk, ch)

    src_r = pad_reshape(src, EPT_PAD, NBLK, SB * CH).reshape(NW, NBLK, SB, CH)
    dst_r = pad_reshape(dst, EPT_PAD, NBLK, SB * CH).reshape(NW, NBLK, SB, CH)
    dst_d = pad_reshape(dst, EPT_PAD_D, NCHUNK_D, CH_D)
    deg_p = deg_call(dst_d)
    y, dis = mm_call(deg_p, x, W)
    part = edge_call(y, src_r, dst_r)
    out = fin_call(part, dis, b)
    return (out, edge_index)
